# baseline (speedup 1.0000x reference)
import functools

import jax
import jax.numpy as jnp
from jax import lax
from jax.experimental import pallas as pl
from jax.experimental.pallas import tpu as pltpu
from jax.experimental.pallas import tpu_sc as plsc

N_CODES = 8192
D = 32
B = 16
HW = 1024
NB = 8192
NCH = N_CODES // NB


def _argmax_body(z_ref, e_ref, ind_ref, rmax, ridx):
    c = pl.program_id(1)

    @pl.when(c == 0)
    def _init():
        rmax[...] = jnp.full((1, HW), -jnp.inf, jnp.float32)
        ridx[...] = jnp.zeros((1, HW), jnp.int32)

    zb = z_ref[0]
    eb = e_ref[...]
    logits = lax.dot_general(eb, zb, (((1,), (0,)), ((), ())),
                             preferred_element_type=jnp.float32)
    m = jnp.max(logits, axis=0, keepdims=True)
    bi = jnp.argmax(logits, axis=0)[None, :].astype(jnp.int32)
    better = m > rmax[...]
    ridx[...] = jnp.where(better, bi + c * NB, ridx[...])
    rmax[...] = jnp.where(better, m, rmax[...])

    @pl.when(c == NCH - 1)
    def _emit():
        ind_ref[0] = ridx[...]


def _argmax_call(z3, embedding):
    return pl.pallas_call(
        _argmax_body,
        grid=(B, NCH),
        in_specs=[
            pl.BlockSpec((1, D, HW), lambda b, c: (b, 0, 0)),
            pl.BlockSpec((NB, D), lambda b, c: (c, 0)),
        ],
        out_specs=pl.BlockSpec((1, 1, HW), lambda b, c: (b, 0, 0)),
        out_shape=jax.ShapeDtypeStruct((B, 1, HW), jnp.int32),
        scratch_shapes=[
            pltpu.VMEM((1, HW), jnp.float32),
            pltpu.VMEM((1, HW), jnp.int32),
        ],
        compiler_params=pltpu.CompilerParams(
            dimension_semantics=("parallel", "arbitrary")),
    )(z3, embedding)


_NW = 32
_BPW = (B * HW) // _NW


@functools.lru_cache(maxsize=1)
def _sc_gather_fn():
    @functools.partial(
        pl.kernel,
        mesh=plsc.VectorSubcoreMesh(core_axis_name="c", subcore_axis_name="s"),
        out_type=jax.ShapeDtypeStruct((B, D, HW, 1), jnp.float32),
        scratch_types=[
            pltpu.VMEM((_BPW,), jnp.int32),
            pltpu.VMEM((_BPW, D), jnp.float32),
            pltpu.SemaphoreType.DMA,
            pltpu.SemaphoreType.DMA,
        ],
        compiler_params=pltpu.CompilerParams(use_tc_tiling_on_sc=False),
    )
    def _sc_gather(table_hbm, idx_hbm, out_hbm, idx_v, rows_v, gsem, ssem):
        wid = lax.axis_index("s") * 2 + lax.axis_index("c")
        base = wid * _BPW
        b = wid // (HW // _BPW)
        hw0 = (wid % (HW // _BPW)) * _BPW
        pltpu.sync_copy(idx_hbm.at[pl.ds(base, _BPW)], idx_v)
        pltpu.async_copy(table_hbm.at[idx_v], rows_v, gsem).wait()
        cps = [pltpu.async_copy(rows_v.at[:, pl.ds(d, 1)],
                                out_hbm.at[b, d, pl.ds(hw0, _BPW)], ssem)
               for d in range(D)]
        for cp in cps:
            cp.wait()

    return _sc_gather


def kernel(z, embedding):
    z3 = z.reshape(B, D, HW)
    ind = _argmax_call(z3, embedding).reshape(-1)
    z_q = _sc_gather_fn()(embedding, ind)
    return (z_q.reshape(B, D, 32, 32), ind)


# speedup vs baseline: 9.8425x; 9.8425x over previous
import functools

import jax
import jax.numpy as jnp
from jax import lax
from jax.experimental import pallas as pl
from jax.experimental.pallas import tpu as pltpu
from jax.experimental.pallas import tpu_sc as plsc

N_CODES = 8192
D = 32
B = 16
HW = 1024
NB = 8192
NCH = N_CODES // NB


def _argmax_body(z_ref, e_ref, ind_ref, rmax, ridx):
    c = pl.program_id(1)

    @pl.when(c == 0)
    def _init():
        rmax[...] = jnp.full((1, HW), -jnp.inf, jnp.float32)
        ridx[...] = jnp.zeros((1, HW), jnp.int32)

    zb = z_ref[0]
    eb = e_ref[...]
    logits = lax.dot_general(eb, zb, (((1,), (0,)), ((), ())),
                             preferred_element_type=jnp.float32)
    m = jnp.max(logits, axis=0, keepdims=True)
    bi = jnp.argmax(logits, axis=0)[None, :].astype(jnp.int32)
    better = m > rmax[...]
    ridx[...] = jnp.where(better, bi + c * NB, ridx[...])
    rmax[...] = jnp.where(better, m, rmax[...])

    @pl.when(c == NCH - 1)
    def _emit():
        ind_ref[0] = ridx[...]


def _argmax_call(z3, embedding):
    return pl.pallas_call(
        _argmax_body,
        grid=(B, NCH),
        in_specs=[
            pl.BlockSpec((1, D, HW), lambda b, c: (b, 0, 0)),
            pl.BlockSpec((NB, D), lambda b, c: (c, 0)),
        ],
        out_specs=pl.BlockSpec((1, 1, HW), lambda b, c: (b, 0, 0)),
        out_shape=jax.ShapeDtypeStruct((B, 1, HW), jnp.int32),
        scratch_shapes=[
            pltpu.VMEM((1, HW), jnp.float32),
            pltpu.VMEM((1, HW), jnp.int32),
        ],
        compiler_params=pltpu.CompilerParams(
            dimension_semantics=("parallel", "arbitrary")),
    )(z3, embedding)


_NW = 32
_BPW = (B * HW) // _NW


@functools.lru_cache(maxsize=1)
def _sc_gather_fn():
    @functools.partial(
        pl.kernel,
        mesh=plsc.VectorSubcoreMesh(core_axis_name="c", subcore_axis_name="s"),
        out_type=jax.ShapeDtypeStruct((B * HW, D), jnp.float32),
        scratch_types=[
            pltpu.VMEM((_BPW,), jnp.int32),
            pltpu.VMEM((_BPW, D), jnp.float32),
            pltpu.SemaphoreType.DMA,
        ],
        compiler_params=pltpu.CompilerParams(use_tc_tiling_on_sc=False),
    )
    def _sc_gather(table_hbm, idx_hbm, out_hbm, idx_v, rows_v, sem):
        wid = lax.axis_index("s") * 2 + lax.axis_index("c")
        base = wid * _BPW
        pltpu.sync_copy(idx_hbm.at[pl.ds(base, _BPW)], idx_v)
        pltpu.async_copy(table_hbm.at[idx_v], rows_v, sem).wait()
        pltpu.sync_copy(rows_v, out_hbm.at[pl.ds(base, _BPW)])

    return _sc_gather


def kernel(z, embedding):
    z3 = z.reshape(B, D, HW)
    ind = _argmax_call(z3, embedding).reshape(-1)
    rows = _sc_gather_fn()(embedding, ind)
    z_q = rows.reshape(B, HW, D).transpose(0, 2, 1).reshape(B, D, 32, 32)
    return (z_q, ind)
